# Initial kernel scaffold; baseline (speedup 1.0000x reference)
import jax
import jax.numpy as jnp
from jax.experimental import pallas as pl


def kernel(x, y):
    raise NotImplementedError("write your pallas kernel here")


# speedup vs baseline: 1.1160x; 1.1160x over previous
import functools

import jax
import jax.numpy as jnp
from jax import lax
from jax.experimental import pallas as pl
from jax.experimental.pallas import tpu as pltpu
from jax.experimental.pallas import tpu_sc as plsc

BATCH = 64
N = 4096
D = 128
NC = 2
NS = 16
NW = NC * NS
ROWS = BATCH * N
RPW = ROWS // NW
C = 128
NCHUNK = RPW // C


def _gather_body(x_hbm, p_hbm, out_hbm, idx_v, rows_v, sem):
    wid = lax.axis_index("s") * NC + lax.axis_index("c")
    base = wid * RPW
    pltpu.sync_copy(p_hbm.at[wid], idx_v)

    def chunk(k, carry):
        off = ((base + k * C) // N) * N
        for t in range(C // 16):
            sl = (k, pl.ds(t * 16, 16))
            idx_v[sl] = idx_v[sl] + off
        pltpu.async_copy(x_hbm.at[idx_v.at[k]], rows_v, sem).wait()
        pltpu.sync_copy(rows_v, out_hbm.at[pl.ds(base + k * C, C)])
        return carry

    lax.fori_loop(0, NCHUNK, chunk, 0)


@jax.jit
def _permute_rows(xf, p3):
    mesh = plsc.VectorSubcoreMesh(
        core_axis_name="c", subcore_axis_name="s", num_cores=NC, num_subcores=NS
    )
    return pl.kernel(
        _gather_body,
        out_type=jax.ShapeDtypeStruct((ROWS, D), jnp.float32),
        mesh=mesh,
        scratch_types=[
            pltpu.VMEM((NCHUNK, C), jnp.int32),
            pltpu.VMEM((C, D), jnp.float32),
            pltpu.SemaphoreType.DMA,
        ],
    )(xf, p3)


def kernel(x, y):
    perm = jax.vmap(
        lambda yi: jax.random.permutation(jax.random.PRNGKey(yi), N)
    )(y)
    p3 = perm.astype(jnp.int32).reshape(NW, NCHUNK, C)
    out = _permute_rows(x.reshape(ROWS, D), p3)
    return out.reshape(BATCH, N, D)


# speedup vs baseline: 1.2845x; 1.1510x over previous
import functools

import jax
import jax.numpy as jnp
from jax import lax
from jax.experimental import pallas as pl
from jax.experimental.pallas import tpu as pltpu
from jax.experimental.pallas import tpu_sc as plsc

BATCH = 64
N = 4096
D = 128
NC = 2
NS = 16
NW = NC * NS
ROWS = BATCH * N
RPW = ROWS // NW
C = 128
NCHUNK = RPW // C


NBUF = 4


def _gather_body(x_hbm, p_hbm, out_hbm, idx_v, *bufs):
    rows = bufs[:NBUF]
    gsem = bufs[NBUF : 2 * NBUF]
    wsem = bufs[2 * NBUF :]
    wid = lax.axis_index("s") * NC + lax.axis_index("c")
    base = wid * RPW
    pltpu.sync_copy(p_hbm.at[wid], idx_v)

    def add_off(k, carry):
        off = ((base + k * C) // N) * N
        for t in range(C // 16):
            sl = (k, pl.ds(t * 16, 16))
            idx_v[sl] = idx_v[sl] + off
        return carry

    lax.fori_loop(0, NCHUNK, add_off, 0)

    def start_gather(k, b):
        pltpu.async_copy(x_hbm.at[idx_v.at[k]], rows[b], gsem[b])

    def wait_gather(b):
        pltpu.make_async_copy(x_hbm.at[pl.ds(0, C)], rows[b], gsem[b]).wait()

    def start_write(k, b):
        pltpu.async_copy(rows[b], out_hbm.at[pl.ds(base + k * C, C)], wsem[b])

    def wait_write(b):
        pltpu.make_async_copy(rows[b], out_hbm.at[pl.ds(0, C)], wsem[b]).wait()

    for b in range(NBUF):
        start_gather(b, b)

    def ring(i, carry):
        for b in range(NBUF):
            k = i * NBUF + b
            wait_gather(b)
            start_write(k, b)
            nk = k + NBUF

            @pl.when(nk < NCHUNK)
            def _():
                wait_write(b)
                start_gather(nk, b)

        return carry

    lax.fori_loop(0, NCHUNK // NBUF, ring, 0)
    for b in range(NBUF):
        wait_write(b)


@jax.jit
def _permute_rows(xf, p3):
    mesh = plsc.VectorSubcoreMesh(
        core_axis_name="c", subcore_axis_name="s", num_cores=NC, num_subcores=NS
    )
    return pl.kernel(
        _gather_body,
        out_type=jax.ShapeDtypeStruct((ROWS, D), jnp.float32),
        mesh=mesh,
        scratch_types=(
            [pltpu.VMEM((NCHUNK, C), jnp.int32)]
            + [pltpu.VMEM((C, D), jnp.float32) for _ in range(NBUF)]
            + [pltpu.SemaphoreType.DMA for _ in range(2 * NBUF)]
        ),
    )(xf, p3)


def kernel(x, y):
    perm = jax.vmap(
        lambda yi: jax.random.permutation(jax.random.PRNGKey(yi), N)
    )(y)
    p3 = perm.astype(jnp.int32).reshape(NW, NCHUNK, C)
    out = _permute_rows(x.reshape(ROWS, D), p3)
    return out.reshape(BATCH, N, D)
